# baseline (speedup 1.0000x reference)
import functools

import jax
import jax.numpy as jnp
from jax import lax
from jax.experimental import pallas as pl
from jax.experimental.pallas import tpu as pltpu
from jax.experimental.pallas import tpu_sc as plsc

D_MODEL = 64
NUM_CORES = 2
NUM_SUBCORES = 16
NUM_WORKERS = NUM_CORES * NUM_SUBCORES
CHUNK = 800


def _emb_body(n_per_w, idx_hbm, table_hbm, out_hbm,
              idx0, idx1, rows0, rows1, g0, g1, w0, w1):
    wid = lax.axis_index("s") * NUM_CORES + lax.axis_index("c")
    base = wid * n_per_w
    nchunks = n_per_w // CHUNK
    idx_v = (idx0, idx1)
    rows_v = (rows0, rows1)
    gsem = (g0, g1)
    wsem = (w0, w1)

    gathers = [None, None]
    writebacks = [None, None]
    for j in range(nchunks):
        b = j % 2
        off = base + j * CHUNK
        if writebacks[b] is not None:
            writebacks[b].wait()
        pltpu.sync_copy(idx_hbm.at[pl.ds(off, CHUNK)], idx_v[b])
        gathers[b] = pltpu.async_copy(table_hbm.at[idx_v[b]], rows_v[b], gsem[b])
        if j >= 1:
            p = 1 - b
            gathers[p].wait()
            prev_off = base + (j - 1) * CHUNK
            writebacks[p] = pltpu.async_copy(
                rows_v[p], out_hbm.at[pl.ds(prev_off, CHUNK)], wsem[p])
    last = (nchunks - 1) % 2
    gathers[last].wait()
    last_off = base + (nchunks - 1) * CHUNK
    writebacks[last] = pltpu.async_copy(
        rows_v[last], out_hbm.at[pl.ds(last_off, CHUNK)], wsem[last])
    writebacks[0].wait()
    writebacks[1].wait()


def kernel(token_ids, emb_table):
    b, s = token_ids.shape
    flat_idx = token_ids.reshape(-1).astype(jnp.int32)
    n = flat_idx.shape[0]
    assert n % (NUM_WORKERS * CHUNK) == 0
    n_per_w = n // NUM_WORKERS

    mesh = plsc.VectorSubcoreMesh(core_axis_name="c", subcore_axis_name="s")
    k = pl.kernel(
        functools.partial(_emb_body, n_per_w),
        mesh=mesh,
        out_type=jax.ShapeDtypeStruct((n, D_MODEL), jnp.float32),
        scratch_types=[
            pltpu.VMEM((CHUNK,), jnp.int32),
            pltpu.VMEM((CHUNK,), jnp.int32),
            pltpu.VMEM((CHUNK, D_MODEL), jnp.float32),
            pltpu.VMEM((CHUNK, D_MODEL), jnp.float32),
            pltpu.SemaphoreType.DMA,
            pltpu.SemaphoreType.DMA,
            pltpu.SemaphoreType.DMA,
            pltpu.SemaphoreType.DMA,
        ],
        compiler_params=pltpu.CompilerParams(use_tc_tiling_on_sc=False),
    )
    out = k(flat_idx, emb_table)
    return out.reshape(b, s, D_MODEL)
